# Initial kernel scaffold; baseline (speedup 1.0000x reference)
import jax
import jax.numpy as jnp
from jax.experimental import pallas as pl


def kernel(stock_features, in_proj_w, in_proj_b, out_proj_w, out_proj_b, ln_gamma, ln_beta):
    raise NotImplementedError("write your pallas kernel here")


# speedup vs baseline: 4.8072x; 4.8072x over previous
import functools

import jax
import jax.numpy as jnp
from jax import lax
from jax.experimental import pallas as pl
from jax.experimental.pallas import tpu as pltpu

_B, _N, _D, _H, _TOPK = 4, 1024, 512, 8, 64
_DH = _D // _H
_NEG = -1e30
_BISECT_ITERS = 16


def _fused_body(x_ref, wqkv_ref, bqkv_ref, wo_ref, bo_ref, g_ref, beta_ref,
                y_ref):
    f32 = jnp.float32
    x = x_ref[0]

    ss = jnp.sum(x * x, axis=-1, keepdims=True)
    norm = jnp.sqrt(ss)
    nrm = x / jnp.maximum(norm, f32(1e-12))
    sim = lax.dot_general(nrm, nrm, (((1,), (1,)), ((), ())),
                          precision=lax.Precision.HIGHEST)

    lo = jnp.full((_N, 1), -1.1, f32)
    hi = jnp.full((_N, 1), 1.1, f32)

    def bis(_, carry):
        lo, hi = carry
        mid = (lo + hi) * f32(0.5)
        cnt = jnp.sum((sim >= mid).astype(f32), axis=-1, keepdims=True)
        pred = cnt >= f32(_TOPK)
        return (jnp.where(pred, mid, lo), jnp.where(pred, hi, mid))

    lo, hi = lax.fori_loop(0, _BISECT_ITERS, bis, (lo, hi))
    thr = lo

    eye = (lax.broadcasted_iota(jnp.int32, (_N, _N), 0) ==
           lax.broadcasted_iota(jnp.int32, (_N, _N), 1))
    allowed = (sim >= thr) | eye

    qkv = (jnp.dot(x, wqkv_ref[...], precision=lax.Precision.HIGHEST)
           + bqkv_ref[...])

    inv_sqrt_dh = f32(1.0) / jnp.sqrt(f32(_DH))
    outs = []
    for h in range(_H):
        qh = qkv[:, h * _DH:(h + 1) * _DH]
        kh = qkv[:, _D + h * _DH:_D + (h + 1) * _DH]
        vh = qkv[:, 2 * _D + h * _DH:2 * _D + (h + 1) * _DH]
        s = lax.dot_general(qh, kh, (((1,), (1,)), ((), ())),
                            precision=lax.Precision.HIGHEST) * inv_sqrt_dh
        s = jnp.where(allowed, s, f32(_NEG))
        m = jnp.max(s, axis=-1, keepdims=True)
        p = jnp.exp(s - m)
        denom = jnp.sum(p, axis=-1, keepdims=True)
        oh = lax.dot_general(p, vh, (((1,), (0,)), ((), ())),
                             precision=lax.Precision.HIGHEST)
        outs.append(oh / denom)
    out = jnp.concatenate(outs, axis=-1)

    att = (jnp.dot(out, wo_ref[...], precision=lax.Precision.HIGHEST)
           + bo_ref[...])

    y = x + att
    mean = jnp.mean(y, axis=-1, keepdims=True)
    yc = y - mean
    var = jnp.mean(yc * yc, axis=-1, keepdims=True)
    y = yc * lax.rsqrt(var + f32(1e-5))
    y_ref[0] = y * g_ref[...] + beta_ref[...]


@jax.jit
def kernel(stock_features, in_proj_w, in_proj_b, out_proj_w, out_proj_b,
           ln_gamma, ln_beta):
    wqkv_t = in_proj_w.T
    wo_t = out_proj_w.T
    bqkv = in_proj_b.reshape(1, 3 * _D)
    bo = out_proj_b.reshape(1, _D)
    g = ln_gamma.reshape(1, _D)
    beta = ln_beta.reshape(1, _D)

    full = lambda shape: pl.BlockSpec(shape, lambda b: (0,) * len(shape))
    return pl.pallas_call(
        _fused_body,
        grid=(_B,),
        in_specs=[
            pl.BlockSpec((1, _N, _D), lambda b: (b, 0, 0)),
            full((_D, 3 * _D)),
            full((1, 3 * _D)),
            full((_D, _D)),
            full((1, _D)),
            full((1, _D)),
            full((1, _D)),
        ],
        out_specs=pl.BlockSpec((1, _N, _D), lambda b: (b, 0, 0)),
        out_shape=jax.ShapeDtypeStruct((_B, _N, _D), jnp.float32),
        compiler_params=pltpu.CompilerParams(
            dimension_semantics=("arbitrary",),
            vmem_limit_bytes=100 * 1024 * 1024,
        ),
    )(stock_features, wqkv_t, bqkv, wo_t, bo, g, beta)


# speedup vs baseline: 14.5849x; 3.0340x over previous
import functools

import jax
import jax.numpy as jnp
from jax import lax
from jax.experimental import pallas as pl
from jax.experimental.pallas import tpu as pltpu

_B, _N, _D, _H, _TOPK = 4, 1024, 512, 8, 64
_DH = _D // _H
_NEG = -1e30
_BISECT_ITERS = 16


def _fused_body(x_ref, wqkv_ref, bqkv_ref, wo_ref, bo_ref, g_ref, beta_ref,
                y_ref):
    f32 = jnp.float32
    x = x_ref[0]

    ss = jnp.sum(x * x, axis=-1, keepdims=True)
    norm = jnp.sqrt(ss)
    nrm = x / jnp.maximum(norm, f32(1e-12))
    bf16 = jnp.bfloat16
    nhi = nrm.astype(bf16)
    nlo = (nrm - nhi.astype(f32)).astype(bf16)
    _nt = (((1,), (1,)), ((), ()))
    sim = (lax.dot_general(nhi, nhi, _nt, preferred_element_type=f32)
           + lax.dot_general(nhi, nlo, _nt, preferred_element_type=f32)
           + lax.dot_general(nlo, nhi, _nt, preferred_element_type=f32))

    lo = jnp.full((_N, 1), -1.1, f32)
    hi = jnp.full((_N, 1), 1.1, f32)

    def bis(_, carry):
        lo, hi = carry
        mid = (lo + hi) * f32(0.5)
        cnt = jnp.sum((sim >= mid).astype(f32), axis=-1, keepdims=True)
        pred = cnt >= f32(_TOPK)
        return (jnp.where(pred, mid, lo), jnp.where(pred, hi, mid))

    lo, hi = lax.fori_loop(0, _BISECT_ITERS, bis, (lo, hi))
    thr = lo

    eye = (lax.broadcasted_iota(jnp.int32, (_N, _N), 0) ==
           lax.broadcasted_iota(jnp.int32, (_N, _N), 1))
    allowed = (sim >= thr) | eye

    bf16 = jnp.bfloat16
    xb = x.astype(bf16)
    qkv = (jnp.dot(xb, wqkv_ref[...], preferred_element_type=f32)
           + bqkv_ref[...])
    qkvb = qkv.astype(bf16)

    inv_sqrt_dh = f32(1.0) / jnp.sqrt(f32(_DH))
    outs = []
    for h in range(_H):
        qh = qkvb[:, h * _DH:(h + 1) * _DH]
        kh = qkvb[:, _D + h * _DH:_D + (h + 1) * _DH]
        vh = qkvb[:, 2 * _D + h * _DH:2 * _D + (h + 1) * _DH]
        s = lax.dot_general(qh, kh, (((1,), (1,)), ((), ())),
                            preferred_element_type=f32) * inv_sqrt_dh
        s = jnp.where(allowed, s, f32(_NEG))
        m = jnp.max(s, axis=-1, keepdims=True)
        p = jnp.exp(s - m)
        denom = jnp.sum(p, axis=-1, keepdims=True)
        oh = lax.dot_general(p.astype(bf16), vh, (((1,), (0,)), ((), ())),
                             preferred_element_type=f32)
        outs.append(oh / denom)
    out = jnp.concatenate(outs, axis=-1)

    att = (jnp.dot(out.astype(bf16), wo_ref[...], preferred_element_type=f32)
           + bo_ref[...])

    y = x + att
    mean = jnp.mean(y, axis=-1, keepdims=True)
    yc = y - mean
    var = jnp.mean(yc * yc, axis=-1, keepdims=True)
    y = yc * lax.rsqrt(var + f32(1e-5))
    y_ref[0] = y * g_ref[...] + beta_ref[...]


@jax.jit
def kernel(stock_features, in_proj_w, in_proj_b, out_proj_w, out_proj_b,
           ln_gamma, ln_beta):
    wqkv_t = in_proj_w.T.astype(jnp.bfloat16)
    wo_t = out_proj_w.T.astype(jnp.bfloat16)
    bqkv = in_proj_b.reshape(1, 3 * _D)
    bo = out_proj_b.reshape(1, _D)
    g = ln_gamma.reshape(1, _D)
    beta = ln_beta.reshape(1, _D)

    full = lambda shape: pl.BlockSpec(shape, lambda b: (0,) * len(shape))
    return pl.pallas_call(
        _fused_body,
        grid=(_B,),
        in_specs=[
            pl.BlockSpec((1, _N, _D), lambda b: (b, 0, 0)),
            full((_D, 3 * _D)),
            full((1, 3 * _D)),
            full((_D, _D)),
            full((1, _D)),
            full((1, _D)),
            full((1, _D)),
        ],
        out_specs=pl.BlockSpec((1, _N, _D), lambda b: (b, 0, 0)),
        out_shape=jax.ShapeDtypeStruct((_B, _N, _D), jnp.float32),
        compiler_params=pltpu.CompilerParams(
            dimension_semantics=("arbitrary",),
            vmem_limit_bytes=100 * 1024 * 1024,
        ),
    )(stock_features, wqkv_t, bqkv, wo_t, bo, g, beta)
